# baseline (speedup 1.0000x reference)
import jax
import jax.numpy as jnp
from jax import lax
from jax.experimental import pallas as pl
from jax.experimental.pallas import tpu as pltpu
from jax.experimental.pallas import tpu_sc as plsc

N_PROPERTIES = 26
N_VALUES = 40000
DIM_EMB = 64
ROWS = 4096 * 20
NC, NS, LANES = 2, 16, 16
NW = NC * NS
CHUNK = 128
FLAT = CHUNK * N_PROPERTIES
ROWS_PER_G = 4
GSZ = ROWS_PER_G * N_PROPERTIES
G_PER_CHUNK = CHUNK // ROWS_PER_G
CHUNKS_PER_W = ROWS // (NW * CHUNK)
VPR = DIM_EMB // LANES
SCALE = 1.0 / N_PROPERTIES


def _sc_body(x2f_hbm, offpat_hbm, table_hbm, out_hbm,
             offv, xchunk, gbuf0, gbuf1, outbuf, sem0, sem1):
  wid = lax.axis_index("s") * NC + lax.axis_index("c")
  gbufs = (gbuf0, gbuf1)
  sems = (sem0, sem1)

  pltpu.sync_copy(offpat_hbm, offv)

  def start(g, p):
    return pltpu.async_copy(
        table_hbm.at[xchunk.at[pl.ds(GSZ * g, GSZ)]], gbufs[p], sems[p])

  def wait(g, p):
    pltpu.make_async_copy(
        table_hbm.at[xchunk.at[pl.ds(GSZ * g, GSZ)]], gbufs[p], sems[p]).wait()

  def accum(p, obase):
    buf = gbufs[p]
    for u in range(ROWS_PER_G):
      for l in range(VPR):
        sl = pl.ds(16 * l, 16)
        v = buf[N_PROPERTIES * u, sl]
        for j in range(1, N_PROPERTIES):
          v = v + buf[N_PROPERTIES * u + j, sl]
        outbuf[obase + u, sl] = v * SCALE

  def chunk_body(t, _):
    c = wid * CHUNKS_PER_W + t
    pltpu.sync_copy(x2f_hbm.at[pl.ds(c * FLAT, FLAT)], xchunk)
    for k in range(FLAT // 16):
      sl = pl.ds(16 * k, 16)
      xchunk[sl] = xchunk[sl] + offv[sl]

    start(0, 0)
    start(1, 1)

    def pair(gg, _):
      a = 2 * gg
      wait(a, 0)
      accum(0, ROWS_PER_G * a)
      start(a + 2, 0)
      wait(a + 1, 1)
      accum(1, ROWS_PER_G * (a + 1))
      start(a + 3, 1)
      return 0

    lax.fori_loop(0, (G_PER_CHUNK - 2) // 2, pair, 0)
    wait(G_PER_CHUNK - 2, 0)
    accum(0, ROWS_PER_G * (G_PER_CHUNK - 2))
    wait(G_PER_CHUNK - 1, 1)
    accum(1, ROWS_PER_G * (G_PER_CHUNK - 1))

    pltpu.sync_copy(outbuf, out_hbm.at[pl.ds(c * CHUNK, CHUNK)])
    return 0

  lax.fori_loop(0, CHUNKS_PER_W, chunk_body, 0)


def _sc_gather_pool(x2f, offpat, table):
  mesh = plsc.VectorSubcoreMesh(core_axis_name="c", subcore_axis_name="s")
  return pl.kernel(
      _sc_body,
      out_type=jax.ShapeDtypeStruct((ROWS, DIM_EMB), jnp.float32),
      mesh=mesh,
      scratch_types=[
          pltpu.VMEM((FLAT,), jnp.int32),
          pltpu.VMEM((FLAT,), jnp.int32),
          pltpu.VMEM((GSZ, DIM_EMB), jnp.float32),
          pltpu.VMEM((GSZ, DIM_EMB), jnp.float32),
          pltpu.VMEM((CHUNK, DIM_EMB), jnp.float32),
          pltpu.SemaphoreType.DMA,
          pltpu.SemaphoreType.DMA,
      ],
      compiler_params=pltpu.CompilerParams(use_tc_tiling_on_sc=False),
  )(x2f, offpat, table)


def _epi_body(pooled_ref, x_ref, mark_ref, emb_ref, pad_ref):
  s = jnp.sum(x_ref[...], axis=1, keepdims=True)
  pad = (s == 0)
  emb_ref[...] = jnp.where(pad, mark_ref[...], pooled_ref[...])
  pad_ref[...] = pad.astype(jnp.int32)


def _tc_epilogue(pooled, x2, mark):
  r_blk = 1024
  grid = (ROWS // r_blk,)
  return pl.pallas_call(
      _epi_body,
      grid=grid,
      in_specs=[
          pl.BlockSpec((r_blk, DIM_EMB), lambda i: (i, 0)),
          pl.BlockSpec((r_blk, N_PROPERTIES), lambda i: (i, 0)),
          pl.BlockSpec((1, DIM_EMB), lambda i: (0, 0)),
      ],
      out_specs=[
          pl.BlockSpec((r_blk, DIM_EMB), lambda i: (i, 0)),
          pl.BlockSpec((r_blk, 1), lambda i: (i, 0)),
      ],
      out_shape=[
          jax.ShapeDtypeStruct((ROWS, DIM_EMB), jnp.float32),
          jax.ShapeDtypeStruct((ROWS, 1), jnp.int32),
      ],
  )(pooled, x2, mark)


@jax.jit
def kernel(x, value_embedding, mark_absent, idx_offset):
  x2 = x.reshape(ROWS, N_PROPERTIES)
  offpat = jnp.tile(idx_offset, CHUNK)
  pooled = _sc_gather_pool(x2.reshape(ROWS * N_PROPERTIES), offpat,
                           value_embedding)
  emb, padi = _tc_epilogue(pooled, x2, mark_absent.reshape(1, DIM_EMB))
  bs, n_roles = x.shape[0], x.shape[1]
  return (emb.reshape(bs, n_roles, DIM_EMB),
          padi.reshape(bs, n_roles) != 0)


# speedup vs baseline: 1.3364x; 1.3364x over previous
import jax
import jax.numpy as jnp
from jax import lax
from jax.experimental import pallas as pl
from jax.experimental.pallas import tpu as pltpu
from jax.experimental.pallas import tpu_sc as plsc

N_PROPERTIES = 26
N_VALUES = 40000
DIM_EMB = 64
ROWS = 4096 * 20
NC, NS, LANES = 2, 16, 16
NW = NC * NS
CHUNK = 128
FLAT = CHUNK * N_PROPERTIES
ROWS_PER_G = 4
GSZ = ROWS_PER_G * N_PROPERTIES
G_PER_CHUNK = CHUNK // ROWS_PER_G
CHUNKS_PER_W = ROWS // (NW * CHUNK)
VPR = DIM_EMB // LANES
SCALE = 1.0 / N_PROPERTIES


def _sc_body(x2f_hbm, offpat_hbm, table_hbm, out_hbm,
             offv, xchunk, gbuf0, gbuf1, outbuf, sem0, sem1):
  wid = lax.axis_index("s") * NC + lax.axis_index("c")
  gbufs = (gbuf0, gbuf1)
  sems = (sem0, sem1)

  pltpu.sync_copy(offpat_hbm, offv)

  def start(g, p):
    return pltpu.async_copy(
        table_hbm.at[xchunk.at[pl.ds(GSZ * g, GSZ)]], gbufs[p], sems[p])

  def wait(g, p):
    pltpu.make_async_copy(
        table_hbm.at[xchunk.at[pl.ds(GSZ * g, GSZ)]], gbufs[p], sems[p]).wait()

  def accum(p, obase):
    buf = gbufs[p]
    for u in range(ROWS_PER_G):
      for l in range(VPR):
        sl = pl.ds(16 * l, 16)
        vs = [buf[N_PROPERTIES * u + j, sl] for j in range(N_PROPERTIES)]
        while len(vs) > 1:
          nxt = [vs[i] + vs[i + 1] for i in range(0, len(vs) - 1, 2)]
          if len(vs) % 2:
            nxt.append(vs[-1])
          vs = nxt
        outbuf[obase + u, sl] = vs[0] * SCALE

  def chunk_body(t, _):
    c = wid * CHUNKS_PER_W + t
    pltpu.sync_copy(x2f_hbm.at[pl.ds(c * FLAT, FLAT)], xchunk)
    for k in range(FLAT // 16):
      sl = pl.ds(16 * k, 16)
      xchunk[sl] = xchunk[sl] + offv[sl]

    start(0, 0)
    start(1, 1)

    def pair(gg, _):
      a = 2 * gg
      wait(a, 0)
      accum(0, ROWS_PER_G * a)
      start(a + 2, 0)
      wait(a + 1, 1)
      accum(1, ROWS_PER_G * (a + 1))
      start(a + 3, 1)
      return 0

    lax.fori_loop(0, (G_PER_CHUNK - 2) // 2, pair, 0)
    wait(G_PER_CHUNK - 2, 0)
    accum(0, ROWS_PER_G * (G_PER_CHUNK - 2))
    wait(G_PER_CHUNK - 1, 1)
    accum(1, ROWS_PER_G * (G_PER_CHUNK - 1))

    pltpu.sync_copy(outbuf, out_hbm.at[pl.ds(c * CHUNK, CHUNK)])
    return 0

  lax.fori_loop(0, CHUNKS_PER_W, chunk_body, 0)


def _sc_gather_pool(x2f, offpat, table):
  mesh = plsc.VectorSubcoreMesh(core_axis_name="c", subcore_axis_name="s")
  return pl.kernel(
      _sc_body,
      out_type=jax.ShapeDtypeStruct((ROWS, DIM_EMB), jnp.float32),
      mesh=mesh,
      scratch_types=[
          pltpu.VMEM((FLAT,), jnp.int32),
          pltpu.VMEM((FLAT,), jnp.int32),
          pltpu.VMEM((GSZ, DIM_EMB), jnp.float32),
          pltpu.VMEM((GSZ, DIM_EMB), jnp.float32),
          pltpu.VMEM((CHUNK, DIM_EMB), jnp.float32),
          pltpu.SemaphoreType.DMA,
          pltpu.SemaphoreType.DMA,
      ],
      compiler_params=pltpu.CompilerParams(use_tc_tiling_on_sc=False),
  )(x2f, offpat, table)


def _epi_body(pooled_ref, x_ref, mark_ref, emb_ref, pad_ref):
  s = jnp.sum(x_ref[...], axis=1, keepdims=True)
  pad = (s == 0)
  emb_ref[...] = jnp.where(pad, mark_ref[...], pooled_ref[...])
  pad_ref[...] = pad.astype(jnp.int32)


def _tc_epilogue(pooled, x2, mark):
  r_blk = 1024
  grid = (ROWS // r_blk,)
  return pl.pallas_call(
      _epi_body,
      grid=grid,
      in_specs=[
          pl.BlockSpec((r_blk, DIM_EMB), lambda i: (i, 0)),
          pl.BlockSpec((r_blk, N_PROPERTIES), lambda i: (i, 0)),
          pl.BlockSpec((1, DIM_EMB), lambda i: (0, 0)),
      ],
      out_specs=[
          pl.BlockSpec((r_blk, DIM_EMB), lambda i: (i, 0)),
          pl.BlockSpec((r_blk, 1), lambda i: (i, 0)),
      ],
      out_shape=[
          jax.ShapeDtypeStruct((ROWS, DIM_EMB), jnp.float32),
          jax.ShapeDtypeStruct((ROWS, 1), jnp.int32),
      ],
  )(pooled, x2, mark)


@jax.jit
def kernel(x, value_embedding, mark_absent, idx_offset):
  x2 = x.reshape(ROWS, N_PROPERTIES)
  offpat = jnp.tile(idx_offset, CHUNK)
  pooled = _sc_gather_pool(x2.reshape(ROWS * N_PROPERTIES), offpat,
                           value_embedding)
  emb, padi = _tc_epilogue(pooled, x2, mark_absent.reshape(1, DIM_EMB))
  bs, n_roles = x.shape[0], x.shape[1]
  return (emb.reshape(bs, n_roles, DIM_EMB),
          padi.reshape(bs, n_roles) != 0)
